# baseline (speedup 1.0000x reference)
import jax
import jax.numpy as jnp
from jax.experimental import pallas as pl
from jax.experimental.pallas import tpu as pltpu

ACQ = 1
B, A, S, C = 64, 4, 4096, 2
LANES = 128
SB = S // LANES
ROWS = S * C // LANES


def _copy_body(in_hbm, out_hbm, sems):
    def cp(b):
        return pltpu.make_async_copy(in_hbm.at[b, ACQ], out_hbm.at[b], sems.at[b])

    for b in range(B):
        cp(b).start()
    for b in range(B):
        cp(b).wait()


_copy = pl.pallas_call(
    _copy_body,
    in_specs=[pl.BlockSpec(memory_space=pl.ANY)],
    out_specs=pl.BlockSpec(memory_space=pl.ANY),
    out_shape=jax.ShapeDtypeStruct((B, ROWS, LANES), jnp.float32),
    scratch_shapes=[pltpu.SemaphoreType.DMA((B,))],
)


@jax.jit
def kernel(inputs):
    x = inputs.reshape(B, A, SB, LANES, C)
    x = x.transpose(0, 1, 2, 4, 3).reshape(B, A, ROWS, LANES)
    out = _copy(x)
    out = out.reshape(B, SB, C, LANES).transpose(0, 1, 3, 2)
    return out.reshape(B, S, C)


# speedup vs baseline: 23.7091x; 23.7091x over previous
import jax
import jax.numpy as jnp
from jax.experimental import pallas as pl
from jax.experimental.pallas import tpu as pltpu

ACQ = 1
B, A, S, C = 64, 4, 4096, 2
LANES = 128
SB = S // LANES
ROWS = S * C // LANES

CH = 8
CB = B // CH


def _copy_body(in_hbm, out_hbm, buf, insems, outsems):
    def in_copy(i):
        return pltpu.make_async_copy(
            in_hbm.at[pl.ds(i * CB, CB), ACQ],
            buf.at[pl.ds(i * CB, CB)],
            insems.at[i],
        )

    def out_copy(i):
        return pltpu.make_async_copy(
            buf.at[pl.ds(i * CB, CB)],
            out_hbm.at[pl.ds(i * CB, CB)],
            outsems.at[i],
        )

    for i in range(CH):
        in_copy(i).start()
    for i in range(CH):
        in_copy(i).wait()
        out_copy(i).start()
    for i in range(CH):
        out_copy(i).wait()


_copy = pl.pallas_call(
    _copy_body,
    in_specs=[pl.BlockSpec(memory_space=pl.ANY)],
    out_specs=pl.BlockSpec(memory_space=pl.ANY),
    out_shape=jax.ShapeDtypeStruct((B, ROWS, LANES), jnp.float32),
    scratch_shapes=[
        pltpu.VMEM((B, ROWS, LANES), jnp.float32),
        pltpu.SemaphoreType.DMA((CH,)),
        pltpu.SemaphoreType.DMA((CH,)),
    ],
)


@jax.jit
def kernel(inputs):
    x = inputs.reshape(B, A, SB, LANES, C)
    x = x.transpose(0, 1, 2, 4, 3).reshape(B, A, ROWS, LANES)
    out = _copy(x)
    out = out.reshape(B, SB, C, LANES).transpose(0, 1, 3, 2)
    return out.reshape(B, S, C)


# speedup vs baseline: 24.4683x; 1.0320x over previous
import jax
import jax.numpy as jnp
from jax.experimental import pallas as pl
from jax.experimental.pallas import tpu as pltpu

ACQ = 1
B, A, S, C = 64, 4, 4096, 2
LANES = 128
SB = S // LANES
ROWS = S * C // LANES

CH = 4
CB = B // CH


def _copy_body(in_hbm, out_hbm, buf, insems, outsems):
    def in_copy(i):
        return pltpu.make_async_copy(
            in_hbm.at[pl.ds(i * CB, CB), ACQ],
            buf.at[pl.ds(i * CB, CB)],
            insems.at[i],
        )

    def out_copy(i):
        return pltpu.make_async_copy(
            buf.at[pl.ds(i * CB, CB)],
            out_hbm.at[pl.ds(i * CB, CB)],
            outsems.at[i],
        )

    for i in range(CH):
        in_copy(i).start()
    for i in range(CH):
        in_copy(i).wait()
        out_copy(i).start()
    for i in range(CH):
        out_copy(i).wait()


_copy = pl.pallas_call(
    _copy_body,
    in_specs=[pl.BlockSpec(memory_space=pl.ANY)],
    out_specs=pl.BlockSpec(memory_space=pl.ANY),
    out_shape=jax.ShapeDtypeStruct((B, ROWS, LANES), jnp.float32),
    scratch_shapes=[
        pltpu.VMEM((B, ROWS, LANES), jnp.float32),
        pltpu.SemaphoreType.DMA((CH,)),
        pltpu.SemaphoreType.DMA((CH,)),
    ],
)


@jax.jit
def kernel(inputs):
    x = inputs.reshape(B, A, SB, LANES, C)
    x = x.transpose(0, 1, 2, 4, 3).reshape(B, A, ROWS, LANES)
    out = _copy(x)
    out = out.reshape(B, SB, C, LANES).transpose(0, 1, 3, 2)
    return out.reshape(B, S, C)
